# baseline (speedup 1.0000x reference)
import functools

import jax
import jax.numpy as jnp
from jax import lax
from jax.experimental import pallas as pl
from jax.experimental.pallas import tpu as pltpu, tpu_sc as plsc

MAX_GOALS = 16384
DEPTH = 16
D = 256
G = 4096

NC = 2
NS = 16
L = 16
NW = NC * NS
QPW = G // NW
C = 8
NCHUNK = QPW // C
NPAIR = NCHUNK // 2
DV = D // L

_mesh = plsc.VectorSubcoreMesh(
    core_axis_name="c", subcore_axis_name="s", num_cores=NC, num_subcores=NS
)


@functools.partial(
    pl.kernel,
    out_type=(
        jax.ShapeDtypeStruct((G, D), jnp.float32),
        jax.ShapeDtypeStruct((G,), jnp.float32),
    ),
    mesh=_mesh,
    scratch_types=[
        pltpu.VMEM((QPW,), jnp.int32),
        pltpu.VMEM((QPW,), jnp.int32),
        pltpu.VMEM((C, DEPTH, D), jnp.float32),
        pltpu.VMEM((C, DEPTH, D), jnp.float32),
        pltpu.VMEM((C, D), jnp.float32),
        pltpu.VMEM((QPW,), jnp.float32),
        pltpu.SemaphoreType.DMA,
        pltpu.SemaphoreType.DMA,
    ],
)
def _tracker(gidx_hbm, cnt_hbm, angles_hbm, sum_hbm, cn_hbm,
             gidx_v, cnt_v, blk0_v, blk1_v, out_v, cn_v, sem0, sem1):
    wid = lax.axis_index("s") * NC + lax.axis_index("c")
    base = wid * QPW

    pltpu.sync_copy(gidx_hbm.at[pl.ds(base, QPW)], gidx_v)
    for t in range(QPW // L):
        g = gidx_v[pl.ds(t * L, L)]
        gidx_v[pl.ds(t * L, L)] = jnp.clip(g, 0, MAX_GOALS - 1)

    pltpu.async_copy(cnt_hbm.at[gidx_v], cnt_v, sem0).wait()

    for t in range(QPW // L):
        nv = jnp.minimum(cnt_v[pl.ds(t * L, L)], DEPTH).astype(jnp.float32)
        cn_v[pl.ds(t * L, L)] = nv * (1.0 / DEPTH)
    pltpu.sync_copy(cn_v, cn_hbm.at[pl.ds(base, QPW)])

    blks = (blk0_v, blk1_v)
    sems = (sem0, sem1)

    def start(ci, b):
        pltpu.async_copy(
            angles_hbm.at[gidx_v.at[pl.ds(ci * C, C)]], blks[b], sems[b]
        )

    start(0, 0)
    start(1, 1)

    def pair_body(ci2, carry):
        n16 = jnp.minimum(cnt_v[pl.ds(ci2 * 2 * C, L)], DEPTH)
        inv16 = 1.0 / jnp.maximum(n16.astype(jnp.float32), 1.0)

        for b in range(2):
            ci = ci2 * 2 + b
            blk_v = blks[b]

            pltpu.make_async_copy(
                angles_hbm.at[gidx_v.at[pl.ds(ci * C, C)]], blk_v, sems[b]
            ).wait()

            for q in range(C):
                n_s = n16[b * C + q]
                inv_b = jnp.full((L,), inv16[b * C + q])

                def row2_body(i, acc, q=q, blk_v=blk_v):
                    return tuple(
                        acc[v]
                        + (
                            blk_v[q, i * 2, pl.ds(v * L, L)]
                            + blk_v[q, i * 2 + 1, pl.ds(v * L, L)]
                        )
                        for v in range(DV)
                    )

                acc0 = tuple(jnp.zeros((L,), jnp.float32) for _ in range(DV))
                acc = lax.fori_loop(0, n_s >> 1, row2_body, acc0)
                jm = jnp.maximum(n_s - 1, 0)
                wodd = jnp.full((L,), (n_s & 1).astype(jnp.float32))
                for v in range(DV):
                    out_v[q, pl.ds(v * L, L)] = (
                        acc[v] + blk_v[q, jm, pl.ds(v * L, L)] * wodd
                    ) * inv_b

            pltpu.sync_copy(out_v, sum_hbm.at[pl.ds(base + ci * C, C)])

            @pl.when(ci + 2 < NCHUNK)
            def _(ci=ci, b=b):
                start(ci + 2, b)

        return carry

    lax.fori_loop(0, NPAIR, pair_body, 0)


def kernel(goal_indices, failed_angles, failed_count):
    summary, count_norm = _tracker(goal_indices, failed_count, failed_angles)
    return summary, count_norm


# speedup vs baseline: 1.6809x; 1.6809x over previous
import functools

import jax
import jax.numpy as jnp
from jax import lax
from jax.experimental import pallas as pl
from jax.experimental.pallas import tpu as pltpu, tpu_sc as plsc

MAX_GOALS = 16384
DEPTH = 16
D = 256
G = 4096

NC = 2
NS = 16
L = 16
NW = NC * NS
QPW = G // NW
C = 8
NCHUNK = QPW // C
NPAIR = NCHUNK // 2
DV = D // L

_mesh = plsc.VectorSubcoreMesh(
    core_axis_name="c", subcore_axis_name="s", num_cores=NC, num_subcores=NS
)


@functools.partial(
    pl.kernel,
    out_type=(
        jax.ShapeDtypeStruct((G, D), jnp.float32),
        jax.ShapeDtypeStruct((G,), jnp.float32),
    ),
    mesh=_mesh,
    scratch_types=[
        pltpu.VMEM((QPW,), jnp.int32),
        pltpu.VMEM((QPW,), jnp.int32),
        pltpu.VMEM((C, DEPTH, D), jnp.float32),
        pltpu.VMEM((C, DEPTH, D), jnp.float32),
        pltpu.VMEM((C, D), jnp.float32),
        pltpu.VMEM((QPW,), jnp.float32),
        pltpu.SemaphoreType.DMA,
        pltpu.SemaphoreType.DMA,
    ],
)
def _tracker(gidx_hbm, cnt_hbm, angles_hbm, sum_hbm, cn_hbm,
             gidx_v, cnt_v, blk0_v, blk1_v, out_v, cn_v, sem0, sem1):
    wid = lax.axis_index("s") * NC + lax.axis_index("c")
    base = wid * QPW

    pltpu.sync_copy(gidx_hbm.at[pl.ds(base, QPW)], gidx_v)
    for t in range(QPW // L):
        g = gidx_v[pl.ds(t * L, L)]
        gidx_v[pl.ds(t * L, L)] = jnp.clip(g, 0, MAX_GOALS - 1)

    pltpu.async_copy(cnt_hbm.at[gidx_v], cnt_v, sem0).wait()

    for t in range(QPW // L):
        nv = jnp.minimum(cnt_v[pl.ds(t * L, L)], DEPTH).astype(jnp.float32)
        cn_v[pl.ds(t * L, L)] = nv * (1.0 / DEPTH)
    pltpu.sync_copy(cn_v, cn_hbm.at[pl.ds(base, QPW)])

    blks = (blk0_v, blk1_v)
    sems = (sem0, sem1)

    def start(ci, b):
        pltpu.async_copy(
            angles_hbm.at[gidx_v.at[pl.ds(ci * C, C)]], blks[b], sems[b]
        )

    start(0, 0)
    start(1, 1)

    def pair_body(ci2, carry):
        n16 = jnp.minimum(cnt_v[pl.ds(ci2 * 2 * C, L)], DEPTH)
        inv16 = 1.0 / jnp.maximum(n16.astype(jnp.float32), 1.0)

        for b in range(2):
            ci = ci2 * 2 + b
            blk_v = blks[b]

            pltpu.make_async_copy(
                angles_hbm.at[gidx_v.at[pl.ds(ci * C, C)]], blk_v, sems[b]
            ).wait()

            for q in range(C):
                n_s = n16[b * C + q]
                inv_b = jnp.full((L,), inv16[b * C + q])

                def row_body(j, acc, q=q, blk_v=blk_v):
                    return tuple(
                        acc[v] + blk_v[q, j, pl.ds(v * L, L)] for v in range(DV)
                    )

                acc0 = tuple(jnp.zeros((L,), jnp.float32) for _ in range(DV))
                acc = lax.fori_loop(0, n_s, row_body, acc0)
                for v in range(DV):
                    out_v[q, pl.ds(v * L, L)] = acc[v] * inv_b

            pltpu.sync_copy(out_v, sum_hbm.at[pl.ds(base + ci * C, C)])

            @pl.when(ci + 2 < NCHUNK)
            def _(ci=ci, b=b):
                start(ci + 2, b)

        return carry

    lax.fori_loop(0, NPAIR, pair_body, 0)


def kernel(goal_indices, failed_angles, failed_count):
    summary, count_norm = _tracker(goal_indices, failed_count, failed_angles)
    return summary, count_norm
